# Initial kernel scaffold; baseline (speedup 1.0000x reference)
import jax
import jax.numpy as jnp
from jax.experimental import pallas as pl


def kernel(input_labels, out_labels, noise, in_embed_w, out_embed_w):
    raise NotImplementedError("write your pallas kernel here")


# speedup vs baseline: 1.3035x; 1.3035x over previous
import functools

import jax
import jax.numpy as jnp
from jax import lax
from jax.experimental import pallas as pl
from jax.experimental.pallas import tpu as pltpu
from jax.experimental.pallas import tpu_sc as plsc

_B = 4096
_V = 100000
_D = 128
_C = 20
_S = 5
_LANES = 16
_CH = 8
_DBLK = _D // _LANES


def _sc_body(in_table, out_table, in_idx, out_idx, noise_idx,
             t_out, ndp_out,
             in_idx_v, out_idx_a, out_idx_b, noise_idx_v,
             in_rows, out_rows_a, out_rows_b, noise_rows,
             t_v, ndp_v, sem, n_workers):
    wid = lax.axis_index("s") * lax.axis_size("c") + lax.axis_index("c")
    per_w = _B // n_workers
    n_iters = per_w // _CH

    def body(k, carry):
        base = wid * per_w + k * _CH

        pltpu.sync_copy(in_idx.at[pl.ds(base, _CH)], in_idx_v)
        pltpu.sync_copy(out_idx.at[pl.ds(base * _C, _CH * _C // 2)], out_idx_a)
        pltpu.sync_copy(out_idx.at[pl.ds(base * _C + _CH * _C // 2,
                                         _CH * _C // 2)], out_idx_b)
        pltpu.sync_copy(noise_idx.at[pl.ds(base * _S, _CH * _S)], noise_idx_v)

        d1 = pltpu.async_copy(in_table.at[in_idx_v], in_rows, sem)
        d2 = pltpu.async_copy(out_table.at[out_idx_a], out_rows_a, sem)
        d3 = pltpu.async_copy(out_table.at[out_idx_b], out_rows_b, sem)
        d4 = pltpu.async_copy(out_table.at[noise_idx_v], noise_rows, sem)
        d1.wait()
        d2.wait()
        d3.wait()
        d4.wait()

        for b in range(_CH):
            ctx_buf = out_rows_a if b < _CH // 2 else out_rows_b
            cbase = (b % (_CH // 2)) * _C
            nd = [None] * _S
            for j in range(_DBLK):
                dsl = pl.ds(j * _LANES, _LANES)
                inv = in_rows[b, dsl]
                acc = ctx_buf[cbase, dsl]
                for c in range(1, _C):
                    acc = acc + ctx_buf[cbase + c, dsl]
                t_v[b, dsl] = inv * acc
                for s in range(_S):
                    prod = inv * noise_rows[b * _S + s, dsl]
                    nd[s] = prod if nd[s] is None else nd[s] + prod
            for s in range(_S):
                ndp_v[b, pl.ds(s * _LANES, _LANES)] = nd[s]

        pltpu.sync_copy(t_v, t_out.at[pl.ds(base, _CH)])
        pltpu.sync_copy(ndp_v, ndp_out.at[pl.ds(base, _CH)])
        return carry

    lax.fori_loop(0, n_iters, body, 0)


def _tc_body(t_ref, ndp_ref, out_ref):
    t = t_ref[...]
    ndp = ndp_ref[...].reshape(_B, _S, _LANES)
    dots = jnp.sum(ndp, axis=-1)
    total = (jnp.sum(jnp.log(jax.nn.sigmoid(t)))
             + jnp.sum(jnp.log(jax.nn.sigmoid(dots))))
    out_ref[...] = jnp.reshape(-total / _B, (1, 1))


def kernel(input_labels, out_labels, noise, in_embed_w, out_embed_w):
    in_idx = input_labels.astype(jnp.int32)
    out_idx = out_labels.astype(jnp.int32).reshape(_B * _C)
    noise_idx = noise.astype(jnp.int32).reshape(_B * _S)

    mesh = plsc.VectorSubcoreMesh(core_axis_name="c", subcore_axis_name="s")
    n_workers = mesh.num_cores * mesh.num_subcores

    sc = pl.kernel(
        functools.partial(_sc_body, n_workers=n_workers),
        out_type=(
            jax.ShapeDtypeStruct((_B, _D), jnp.float32),
            jax.ShapeDtypeStruct((_B, _S * _LANES), jnp.float32),
        ),
        mesh=mesh,
        scratch_types=[
            pltpu.VMEM((_CH,), jnp.int32),
            pltpu.VMEM((_CH * _C // 2,), jnp.int32),
            pltpu.VMEM((_CH * _C // 2,), jnp.int32),
            pltpu.VMEM((_CH * _S,), jnp.int32),
            pltpu.VMEM((_CH, _D), jnp.float32),
            pltpu.VMEM((_CH * _C // 2, _D), jnp.float32),
            pltpu.VMEM((_CH * _C // 2, _D), jnp.float32),
            pltpu.VMEM((_CH * _S, _D), jnp.float32),
            pltpu.VMEM((_CH, _D), jnp.float32),
            pltpu.VMEM((_CH, _S * _LANES), jnp.float32),
            pltpu.SemaphoreType.DMA,
        ],
    )
    t, ndp = sc(in_embed_w, out_embed_w, in_idx, out_idx, noise_idx)

    loss = pl.pallas_call(
        _tc_body,
        out_shape=jax.ShapeDtypeStruct((1, 1), jnp.float32),
    )(t, ndp)
    return loss[0, 0]
